# baseline (speedup 1.0000x reference)
import functools

import jax
import jax.numpy as jnp
from jax import lax
from jax.experimental import pallas as pl
from jax.experimental.pallas import tpu as pltpu
from jax.experimental.pallas import tpu_sc as plsc

INC = 32
OUTC = 64
INN = 8192
OUTN = 64
MAXD = 8
NB = 128
USED = OUTN * MAXD
PADG = MAXD + 1

NSC = 64
NTC = NB - NSC
BLK = 32
HALF = BLK // 2

NWORK = 32
B_PER_W = NSC // NWORK


def _sc_body(x_hbm, nf_hbm, out_hbm, xv, wv, pv, h2v):
    cid = lax.axis_index("c")
    sid = lax.axis_index("s")
    wid = sid * 2 + cid
    iota = lax.broadcasted_iota(jnp.int32, (16,), 0)
    pidx = iota + (iota // MAXD)

    pltpu.sync_copy(nf_hbm.at[:, pl.ds(0, USED)], wv)

    def do_batch(t, _):
        b = NTC + wid * B_PER_W + t
        pltpu.sync_copy(x_hbm.at[b, :, pl.ds(0, USED)], xv)

        def do_row(r, _):
            def pass1(g, _):
                xc = xv[r, pl.ds(g * 16, 16)]
                wc = wv[r, pl.ds(g * 16, 16)]
                plsc.store_scatter(pv, [pidx + g * (2 * PADG)], xc * wc)
                return 0

            lax.fori_loop(0, USED // 16, pass1, 0)

            def pass2(q, _):
                base = PADG * 16 * q + PADG * iota
                acc = plsc.load_gather(pv, [base])
                for d in range(1, MAXD):
                    acc = jnp.maximum(acc, plsc.load_gather(pv, [base + d]))
                h2v[0, r, pl.ds(q * 16, 16)] = acc
                return 0

            lax.fori_loop(0, OUTN // 16, pass2, 0)
            return 0

        lax.fori_loop(0, INC, do_row, 0)
        pltpu.sync_copy(h2v, out_hbm.at[pl.ds(wid * B_PER_W + t, 1)])
        return 0

    lax.fori_loop(0, B_PER_W, do_batch, 0)


def _sc_stage(x, nf_weight):
    mesh = plsc.VectorSubcoreMesh(core_axis_name="c", subcore_axis_name="s")
    f = functools.partial(
        pl.kernel,
        mesh=mesh,
        compiler_params=pltpu.CompilerParams(needs_layout_passes=False),
        out_type=jax.ShapeDtypeStruct((NSC, INC, OUTN), jnp.float32),
        scratch_types=[
            pltpu.VMEM((INC, USED), jnp.float32),
            pltpu.VMEM((INC, USED), jnp.float32),
            pltpu.VMEM((OUTN * PADG,), jnp.float32),
            pltpu.VMEM((1, INC, OUTN), jnp.float32),
        ],
    )(_sc_body)
    return f(x, nf_weight)


def _grouped_max(h):
    t = jnp.maximum(h, pltpu.roll(h, shift=USED - 1, axis=2))
    t = jnp.maximum(t, pltpu.roll(t, shift=USED - 2, axis=2))
    t = jnp.maximum(t, pltpu.roll(t, shift=USED - 4, axis=2))
    return t


def _transform(m, ft, bias):
    out = lax.dot_general(
        m, ft,
        dimension_numbers=(((1,), (0,)), ((), ())),
        preferred_element_type=jnp.float32,
    )
    return out.transpose(0, 2, 1) + bias[None, :, :]


def _tc_kernel(xa_ref, xb_ref, nf_ref, ft_ref, bias_ref, out_ref):
    nf = nf_ref[...]
    t = jnp.concatenate(
        [_grouped_max(xa_ref[...] * nf[None, :, :]),
         _grouped_max(xb_ref[...] * nf[None, :, :])],
        axis=0,
    )
    jj = lax.broadcasted_iota(jnp.int32, (USED, OUTN), 0)
    oo = lax.broadcasted_iota(jnp.int32, (USED, OUTN), 1)
    sel = (jj == oo * MAXD).astype(jnp.float32)
    m = lax.dot_general(
        t, sel,
        dimension_numbers=(((2,), (0,)), ((), ())),
        preferred_element_type=jnp.float32,
    )
    out_ref[...] = _transform(m, ft_ref[...], bias_ref[...])


def _tc_stage(x, nf_weight, ft_weight, bias):
    grid = (NTC // BLK,)
    return pl.pallas_call(
        _tc_kernel,
        grid=grid,
        in_specs=[
            pl.BlockSpec((HALF, INC, USED), lambda j: (2 * j, 0, 0)),
            pl.BlockSpec((HALF, INC, USED), lambda j: (2 * j + 1, 0, 0)),
            pl.BlockSpec((INC, USED), lambda j: (0, 0)),
            pl.BlockSpec((INC, OUTC), lambda j: (0, 0)),
            pl.BlockSpec((OUTC, OUTN), lambda j: (0, 0)),
        ],
        out_specs=pl.BlockSpec((BLK, OUTC, OUTN), lambda j: (j, 0, 0)),
        out_shape=jax.ShapeDtypeStruct((NTC, OUTC, OUTN), jnp.float32),
    )(x, x, nf_weight, ft_weight, bias)


def _mm_kernel(m_ref, ft_ref, bias_ref, out_ref):
    out_ref[...] = _transform(m_ref[...], ft_ref[...], bias_ref[...])


def _mm_stage(h2, ft_weight, bias):
    return pl.pallas_call(
        _mm_kernel,
        out_shape=jax.ShapeDtypeStruct((NSC, OUTC, OUTN), jnp.float32),
    )(h2, ft_weight, bias)


def kernel(x, nf_weight, ft_weight, bias):
    h2_sc = _sc_stage(x, nf_weight)
    out_tc = _tc_stage(x, nf_weight, ft_weight, bias)
    out_sc = _mm_stage(h2_sc, ft_weight, bias)
    return jnp.concatenate([out_tc, out_sc], axis=0)


# speedup vs baseline: 1.0033x; 1.0033x over previous
import functools

import jax
import jax.numpy as jnp
from jax import lax
from jax.experimental import pallas as pl
from jax.experimental.pallas import tpu as pltpu
from jax.experimental.pallas import tpu_sc as plsc

INC = 32
OUTC = 64
INN = 8192
OUTN = 64
MAXD = 8
NB = 128
USED = OUTN * MAXD
PADG = MAXD + 1

NSC = 64
NTC = NB - NSC
BLK = 32
HALF = BLK // 2

NWORK = 32
B_PER_W = NSC // NWORK


def _sc_body(x_hbm, nf_hbm, out_hbm, xv0, xv1, wv, pv, h2v0, h2v1,
             sem0, sem1, osem):
    cid = lax.axis_index("c")
    sid = lax.axis_index("s")
    wid = sid * 2 + cid
    iota = lax.broadcasted_iota(jnp.int32, (16,), 0)
    pidx = iota + (iota // MAXD)

    xvs = (xv0, xv1)
    h2vs = (h2v0, h2v1)
    sems = (sem0, sem1)

    copies = []
    for t in range(B_PER_W):
        b = NTC + wid * B_PER_W + t
        copies.append(
            pltpu.async_copy(x_hbm.at[b, :, pl.ds(0, USED)], xvs[t], sems[t]))
    pltpu.sync_copy(nf_hbm.at[:, pl.ds(0, USED)], wv)

    out_copies = []
    for t in range(B_PER_W):
        copies[t].wait()
        xv = xvs[t]
        h2v = h2vs[t]

        def do_row(r, _):
            for g in range(USED // 16):
                xc = xv[r, pl.ds(g * 16, 16)]
                wc = wv[r, pl.ds(g * 16, 16)]
                plsc.store_scatter(pv, [pidx + g * (2 * PADG)], xc * wc)
            for q in range(OUTN // 16):
                base = PADG * 16 * q + PADG * iota
                acc = plsc.load_gather(pv, [base])
                for d in range(1, MAXD):
                    acc = jnp.maximum(acc, plsc.load_gather(pv, [base + d]))
                h2v[0, r, pl.ds(q * 16, 16)] = acc
            return 0

        lax.fori_loop(0, INC, do_row, 0)
        out_copies.append(pltpu.async_copy(
            h2v, out_hbm.at[pl.ds(wid * B_PER_W + t, 1)], osem))
    for cp in out_copies:
        cp.wait()


def _sc_stage(x, nf_weight):
    mesh = plsc.VectorSubcoreMesh(core_axis_name="c", subcore_axis_name="s")
    f = functools.partial(
        pl.kernel,
        mesh=mesh,
        compiler_params=pltpu.CompilerParams(needs_layout_passes=False),
        out_type=jax.ShapeDtypeStruct((NSC, INC, OUTN), jnp.float32),
        scratch_types=[
            pltpu.VMEM((INC, USED), jnp.float32),
            pltpu.VMEM((INC, USED), jnp.float32),
            pltpu.VMEM((INC, USED), jnp.float32),
            pltpu.VMEM((OUTN * PADG,), jnp.float32),
            pltpu.VMEM((1, INC, OUTN), jnp.float32),
            pltpu.VMEM((1, INC, OUTN), jnp.float32),
            pltpu.SemaphoreType.DMA,
            pltpu.SemaphoreType.DMA,
            pltpu.SemaphoreType.DMA,
        ],
    )(_sc_body)
    return f(x, nf_weight)


def _grouped_max(h):
    t = jnp.maximum(h, pltpu.roll(h, shift=USED - 1, axis=2))
    t = jnp.maximum(t, pltpu.roll(t, shift=USED - 2, axis=2))
    t = jnp.maximum(t, pltpu.roll(t, shift=USED - 4, axis=2))
    return t


def _transform(m, ft, bias):
    out = lax.dot_general(
        m, ft,
        dimension_numbers=(((1,), (0,)), ((), ())),
        preferred_element_type=jnp.float32,
    )
    return out.transpose(0, 2, 1) + bias[None, :, :]


def _tc_kernel(xa_ref, xb_ref, nf_ref, ft_ref, bias_ref, out_ref):
    nf = nf_ref[...]
    t = jnp.concatenate(
        [_grouped_max(xa_ref[...] * nf[None, :, :]),
         _grouped_max(xb_ref[...] * nf[None, :, :])],
        axis=0,
    )
    jj = lax.broadcasted_iota(jnp.int32, (USED, OUTN), 0)
    oo = lax.broadcasted_iota(jnp.int32, (USED, OUTN), 1)
    sel = (jj == oo * MAXD).astype(jnp.float32)
    m = lax.dot_general(
        t, sel,
        dimension_numbers=(((2,), (0,)), ((), ())),
        preferred_element_type=jnp.float32,
    )
    out_ref[...] = _transform(m, ft_ref[...], bias_ref[...])


def _tc_stage(x, nf_weight, ft_weight, bias):
    grid = (NTC // BLK,)
    return pl.pallas_call(
        _tc_kernel,
        grid=grid,
        in_specs=[
            pl.BlockSpec((HALF, INC, USED), lambda j: (2 * j, 0, 0)),
            pl.BlockSpec((HALF, INC, USED), lambda j: (2 * j + 1, 0, 0)),
            pl.BlockSpec((INC, USED), lambda j: (0, 0)),
            pl.BlockSpec((INC, OUTC), lambda j: (0, 0)),
            pl.BlockSpec((OUTC, OUTN), lambda j: (0, 0)),
        ],
        out_specs=pl.BlockSpec((BLK, OUTC, OUTN), lambda j: (j, 0, 0)),
        out_shape=jax.ShapeDtypeStruct((NTC, OUTC, OUTN), jnp.float32),
    )(x, x, nf_weight, ft_weight, bias)


def _mm_kernel(m_ref, ft_ref, bias_ref, out_ref):
    out_ref[...] = _transform(m_ref[...], ft_ref[...], bias_ref[...])


def _mm_stage(h2, ft_weight, bias):
    return pl.pallas_call(
        _mm_kernel,
        out_shape=jax.ShapeDtypeStruct((NSC, OUTC, OUTN), jnp.float32),
    )(h2, ft_weight, bias)


def kernel(x, nf_weight, ft_weight, bias):
    h2_sc = _sc_stage(x, nf_weight)
    out_tc = _tc_stage(x, nf_weight, ft_weight, bias)
    out_sc = _mm_stage(h2_sc, ft_weight, bias)
    return jnp.concatenate([out_tc, out_sc], axis=0)


# speedup vs baseline: 1.0672x; 1.0636x over previous
import functools

import jax
import jax.numpy as jnp
from jax import lax
from jax.experimental import pallas as pl
from jax.experimental.pallas import tpu as pltpu
from jax.experimental.pallas import tpu_sc as plsc

INC = 32
OUTC = 64
INN = 8192
OUTN = 64
MAXD = 8
NB = 128
USED = OUTN * MAXD
PADG = MAXD + 1

NSC = 32
NTC = NB - NSC
BLK = 32
HALF = BLK // 2

NWORK = 16
B_PER_W = NSC // NWORK


def _sc_body(x_hbm, nf_hbm, out_hbm, xv0, xv1, wv, pv, h2v0, h2v1,
             sem0, sem1, osem):
    cid = lax.axis_index("c")
    sid = lax.axis_index("s")
    wid = sid
    iota = lax.broadcasted_iota(jnp.int32, (16,), 0)
    pidx = iota + (iota // MAXD)

    xvs = (xv0, xv1)
    h2vs = (h2v0, h2v1)
    sems = (sem0, sem1)

    copies = []
    for t in range(B_PER_W):
        b = NTC + wid * B_PER_W + t
        copies.append(
            pltpu.async_copy(x_hbm.at[b, :, pl.ds(0, USED)], xvs[t], sems[t]))
    pltpu.sync_copy(nf_hbm.at[:, pl.ds(0, USED)], wv)

    out_copies = []
    for t in range(B_PER_W):
        copies[t].wait()
        xv = xvs[t]
        h2v = h2vs[t]

        def do_row(r, _):
            for g in range(USED // 16):
                xc = xv[r, pl.ds(g * 16, 16)]
                wc = wv[r, pl.ds(g * 16, 16)]
                plsc.store_scatter(pv, [pidx + g * (2 * PADG)], xc * wc)
            for q in range(OUTN // 16):
                base = PADG * 16 * q + PADG * iota
                acc = plsc.load_gather(pv, [base])
                for d in range(1, MAXD):
                    acc = jnp.maximum(acc, plsc.load_gather(pv, [base + d]))
                h2v[0, r, pl.ds(q * 16, 16)] = acc
            return 0

        lax.fori_loop(0, INC, do_row, 0)
        out_copies.append(pltpu.async_copy(
            h2v, out_hbm.at[pl.ds(wid * B_PER_W + t, 1)], osem))
    for cp in out_copies:
        cp.wait()


def _sc_stage(x, nf_weight):
    mesh = plsc.VectorSubcoreMesh(core_axis_name="c", subcore_axis_name="s", num_cores=1)
    f = functools.partial(
        pl.kernel,
        mesh=mesh,
        compiler_params=pltpu.CompilerParams(needs_layout_passes=False),
        out_type=jax.ShapeDtypeStruct((NSC, INC, OUTN), jnp.float32),
        scratch_types=[
            pltpu.VMEM((INC, USED), jnp.float32),
            pltpu.VMEM((INC, USED), jnp.float32),
            pltpu.VMEM((INC, USED), jnp.float32),
            pltpu.VMEM((OUTN * PADG,), jnp.float32),
            pltpu.VMEM((1, INC, OUTN), jnp.float32),
            pltpu.VMEM((1, INC, OUTN), jnp.float32),
            pltpu.SemaphoreType.DMA,
            pltpu.SemaphoreType.DMA,
            pltpu.SemaphoreType.DMA,
        ],
    )(_sc_body)
    return f(x, nf_weight)


def _grouped_max(h):
    t = jnp.maximum(h, pltpu.roll(h, shift=USED - 1, axis=2))
    t = jnp.maximum(t, pltpu.roll(t, shift=USED - 2, axis=2))
    t = jnp.maximum(t, pltpu.roll(t, shift=USED - 4, axis=2))
    return t


def _transform(m, ft, bias):
    out = lax.dot_general(
        m, ft,
        dimension_numbers=(((1,), (0,)), ((), ())),
        preferred_element_type=jnp.float32,
    )
    return out.transpose(0, 2, 1) + bias[None, :, :]


def _tc_kernel(xa_ref, xb_ref, nf_ref, ft_ref, bias_ref, out_ref):
    nf = nf_ref[...]
    t = jnp.concatenate(
        [_grouped_max(xa_ref[...] * nf[None, :, :]),
         _grouped_max(xb_ref[...] * nf[None, :, :])],
        axis=0,
    )
    jj = lax.broadcasted_iota(jnp.int32, (USED, OUTN), 0)
    oo = lax.broadcasted_iota(jnp.int32, (USED, OUTN), 1)
    sel = (jj == oo * MAXD).astype(jnp.float32)
    m = lax.dot_general(
        t, sel,
        dimension_numbers=(((2,), (0,)), ((), ())),
        preferred_element_type=jnp.float32,
    )
    out_ref[...] = _transform(m, ft_ref[...], bias_ref[...])


def _tc_stage(x, nf_weight, ft_weight, bias):
    grid = (NTC // BLK,)
    return pl.pallas_call(
        _tc_kernel,
        grid=grid,
        in_specs=[
            pl.BlockSpec((HALF, INC, USED), lambda j: (2 * j, 0, 0)),
            pl.BlockSpec((HALF, INC, USED), lambda j: (2 * j + 1, 0, 0)),
            pl.BlockSpec((INC, USED), lambda j: (0, 0)),
            pl.BlockSpec((INC, OUTC), lambda j: (0, 0)),
            pl.BlockSpec((OUTC, OUTN), lambda j: (0, 0)),
        ],
        out_specs=pl.BlockSpec((BLK, OUTC, OUTN), lambda j: (j, 0, 0)),
        out_shape=jax.ShapeDtypeStruct((NTC, OUTC, OUTN), jnp.float32),
    )(x, x, nf_weight, ft_weight, bias)


def _mm_kernel(m_ref, ft_ref, bias_ref, out_ref):
    out_ref[...] = _transform(m_ref[...], ft_ref[...], bias_ref[...])


def _mm_stage(h2, ft_weight, bias):
    return pl.pallas_call(
        _mm_kernel,
        out_shape=jax.ShapeDtypeStruct((NSC, OUTC, OUTN), jnp.float32),
    )(h2, ft_weight, bias)


def kernel(x, nf_weight, ft_weight, bias):
    h2_sc = _sc_stage(x, nf_weight)
    out_tc = _tc_stage(x, nf_weight, ft_weight, bias)
    out_sc = _mm_stage(h2_sc, ft_weight, bias)
    return jnp.concatenate([out_tc, out_sc], axis=0)


# speedup vs baseline: 3.2838x; 3.0772x over previous
import jax
import jax.numpy as jnp
from jax.experimental import pallas as pl
from jax.experimental.pallas import tpu as pltpu

INC = 32
OUTC = 64
INN = 8192
OUTN = 64
MAXD = 8
NB = 128
USED = OUTN * MAXD
BLK = 32
QUAR = BLK // 4


def _grouped_max(h):
    t = jnp.maximum(h, pltpu.roll(h, shift=USED - 1, axis=2))
    t = jnp.maximum(t, pltpu.roll(t, shift=USED - 2, axis=2))
    t = jnp.maximum(t, pltpu.roll(t, shift=USED - 4, axis=2))
    return t


def _fgl_kernel(xa_ref, xb_ref, xc_ref, xd_ref, nf_ref, ft_ref, bias_ref, out_ref):
    nf = nf_ref[...]
    t = jnp.concatenate(
        [_grouped_max(xa_ref[...] * nf[None, :, :]),
         _grouped_max(xb_ref[...] * nf[None, :, :]),
         _grouped_max(xc_ref[...] * nf[None, :, :]),
         _grouped_max(xd_ref[...] * nf[None, :, :])],
        axis=0,
    )
    jj = jax.lax.broadcasted_iota(jnp.int32, (USED, OUTN), 0)
    oo = jax.lax.broadcasted_iota(jnp.int32, (USED, OUTN), 1)
    sel = (jj == oo * MAXD).astype(jnp.float32)
    m = jax.lax.dot_general(
        t, sel,
        dimension_numbers=(((2,), (0,)), ((), ())),
        preferred_element_type=jnp.float32,
    )
    ft = ft_ref[...]
    out = jax.lax.dot_general(
        m, ft,
        dimension_numbers=(((1,), (0,)), ((), ())),
        preferred_element_type=jnp.float32,
    )
    out = out.transpose(0, 2, 1)
    out_ref[...] = out + bias_ref[...][None, :, :]


def kernel(x, nf_weight, ft_weight, bias):
    grid = (NB // BLK,)
    return pl.pallas_call(
        _fgl_kernel,
        grid=grid,
        in_specs=[
            pl.BlockSpec((QUAR, INC, USED), lambda j: (4 * j, 0, 0)),
            pl.BlockSpec((QUAR, INC, USED), lambda j: (4 * j + 1, 0, 0)),
            pl.BlockSpec((QUAR, INC, USED), lambda j: (4 * j + 2, 0, 0)),
            pl.BlockSpec((QUAR, INC, USED), lambda j: (4 * j + 3, 0, 0)),
            pl.BlockSpec((INC, USED), lambda j: (0, 0)),
            pl.BlockSpec((INC, OUTC), lambda j: (0, 0)),
            pl.BlockSpec((OUTC, OUTN), lambda j: (0, 0)),
        ],
        out_specs=pl.BlockSpec((BLK, OUTC, OUTN), lambda j: (j, 0, 0)),
        out_shape=jax.ShapeDtypeStruct((NB, OUTC, OUTN), jnp.float32),
    )(x, x, x, x, nf_weight, ft_weight, bias)


# speedup vs baseline: 3.3699x; 1.0262x over previous
import jax
import jax.numpy as jnp
from jax.experimental import pallas as pl
from jax.experimental.pallas import tpu as pltpu

INC = 32
OUTC = 64
INN = 8192
OUTN = 64
MAXD = 8
NB = 128
USED = OUTN * MAXD
BLK = 32
HALF = BLK // 2


def _grouped_max(h):
    t = jnp.maximum(h, pltpu.roll(h, shift=USED - 1, axis=2))
    t = jnp.maximum(t, pltpu.roll(t, shift=USED - 2, axis=2))
    t = jnp.maximum(t, pltpu.roll(t, shift=USED - 4, axis=2))
    return t


def _fgl_kernel(xa_ref, xb_ref, nf_ref, ft_ref, bias_ref, out_ref):
    nf = nf_ref[...]
    t = jnp.concatenate(
        [_grouped_max(xa_ref[...] * nf[None, :, :]),
         _grouped_max(xb_ref[...] * nf[None, :, :])],
        axis=0,
    )
    jj = jax.lax.broadcasted_iota(jnp.int32, (USED, OUTN), 0)
    oo = jax.lax.broadcasted_iota(jnp.int32, (USED, OUTN), 1)
    sel = (jj == oo * MAXD).astype(jnp.float32)
    m = jax.lax.dot_general(
        t, sel,
        dimension_numbers=(((2,), (0,)), ((), ())),
        preferred_element_type=jnp.float32,
    )
    ft = ft_ref[...]
    out = jax.lax.dot_general(
        m, ft,
        dimension_numbers=(((1,), (0,)), ((), ())),
        preferred_element_type=jnp.float32,
    )
    out = out.transpose(0, 2, 1)
    out_ref[...] = out + bias_ref[...][None, :, :]


def kernel(x, nf_weight, ft_weight, bias):
    grid = (NB // BLK,)
    return pl.pallas_call(
        _fgl_kernel,
        grid=grid,
        in_specs=[
            pl.BlockSpec((HALF, INC, USED), lambda j: (2 * j, 0, 0)),
            pl.BlockSpec((HALF, INC, USED), lambda j: (2 * j + 1, 0, 0)),
            pl.BlockSpec((INC, USED), lambda j: (0, 0)),
            pl.BlockSpec((INC, OUTC), lambda j: (0, 0)),
            pl.BlockSpec((OUTC, OUTN), lambda j: (0, 0)),
        ],
        out_specs=pl.BlockSpec((BLK, OUTC, OUTN), lambda j: (j, 0, 0)),
        out_shape=jax.ShapeDtypeStruct((NB, OUTC, OUTN), jnp.float32),
    )(x, x, nf_weight, ft_weight, bias)
